# baseline (speedup 1.0000x reference)
import functools

import jax
import jax.numpy as jnp
from jax import lax
from jax.experimental import pallas as pl
from jax.experimental.pallas import tpu as pltpu
from jax.experimental.pallas import tpu_sc as plsc

B = 16384
D = 64
NUM_REL = 1000

_info = plsc.get_sparse_core_info()
NC = _info.num_cores
NS = _info.num_subcores
L = _info.num_lanes
NW = NC * NS
BPW = B // NW
CH = 256
NCH = BPW // CH


def _make_sc_kernel():
    mesh = plsc.VectorSubcoreMesh(core_axis_name="c", subcore_axis_name="s")

    @functools.partial(
        pl.kernel,
        mesh=mesh,
        compiler_params=pltpu.CompilerParams(needs_layout_passes=False),
        out_type=jax.ShapeDtypeStruct((B,), jnp.float32),
        scratch_types=[
            pltpu.VMEM((NUM_REL * D,), jnp.float32),
            pltpu.VMEM((BPW,), jnp.int32),
            pltpu.VMEM((CH * D,), jnp.float32),
            pltpu.VMEM((CH * D,), jnp.float32),
            pltpu.VMEM((BPW,), jnp.float32),
        ],
    )
    def sc_kernel(s_hbm, o_hbm, rel_hbm, table_hbm, out_hbm,
                  table_v, idx_v, s_v, o_v, out_v):
        wid = lax.axis_index("s") * NC + lax.axis_index("c")
        base = wid * BPW
        lanes = lax.iota(jnp.int32, L)

        pltpu.sync_copy(table_hbm, table_v)
        pltpu.sync_copy(rel_hbm.at[pl.ds(base, BPW)], idx_v)

        for c in range(NCH):
            off = (base + c * CH) * D
            pltpu.sync_copy(s_hbm.at[pl.ds(off, CH * D)], s_v)
            pltpu.sync_copy(o_hbm.at[pl.ds(off, CH * D)], o_v)

            def group_body(g, _, c=c):
                rel16 = idx_v[pl.ds(c * CH + g * L, L)]
                rel_base = rel16 * D
                row_base = (g * L + lanes) * D
                acc = jnp.zeros((L,), jnp.float32)
                for d in range(D):
                    sv = plsc.load_gather(s_v, [row_base + d])
                    rv = plsc.load_gather(table_v, [rel_base + d])
                    ov = plsc.load_gather(o_v, [row_base + d])
                    acc = acc + sv * rv * ov
                out_v[pl.ds(c * CH + g * L, L)] = acc
                return 0

            lax.fori_loop(0, CH // L, group_body, 0)

        pltpu.sync_copy(out_v, out_hbm.at[pl.ds(base, BPW)])

    return sc_kernel


_sc_kernel = _make_sc_kernel()


def kernel(subject_embeddings, object_embeddings, relations, relation_weight):
    scores = _sc_kernel(subject_embeddings.reshape(B * D),
                        object_embeddings.reshape(B * D),
                        relations.astype(jnp.int32),
                        relation_weight.reshape(NUM_REL * D))
    return scores.reshape(B, 1)


# speedup vs baseline: 1.4678x; 1.4678x over previous
import functools

import jax
import jax.numpy as jnp
from jax import lax
from jax.experimental import pallas as pl
from jax.experimental.pallas import tpu as pltpu
from jax.experimental.pallas import tpu_sc as plsc

B = 16384
D = 64
NUM_REL = 1000

_info = plsc.get_sparse_core_info()
NC = _info.num_cores
NS = _info.num_subcores
L = _info.num_lanes
NW = NC * NS
BPW = B // NW
CH = 128
NCH = BPW // CH
KD = D // L


def _make_sc_kernel():
    mesh = plsc.VectorSubcoreMesh(core_axis_name="c", subcore_axis_name="s")

    @functools.partial(
        pl.kernel,
        mesh=mesh,
        compiler_params=pltpu.CompilerParams(needs_layout_passes=False,
                                             use_tc_tiling_on_sc=False),
        out_type=jax.ShapeDtypeStruct((B,), jnp.float32),
        scratch_types=[
            pltpu.VMEM((CH,), jnp.int32),
            pltpu.VMEM((CH * D,), jnp.float32),
            pltpu.VMEM((CH * D,), jnp.float32),
            pltpu.VMEM((CH, D), jnp.float32),
            pltpu.VMEM((BPW,), jnp.float32),
            pltpu.SemaphoreType.DMA,
        ],
    )
    def sc_kernel(s_hbm, o_hbm, rel_hbm, table_hbm, out_hbm,
                  idx_v, s_v, o_v, r_v, out_v, sem):
        wid = lax.axis_index("s") * NC + lax.axis_index("c")
        base = wid * BPW
        lanes = lax.iota(jnp.int32, L)

        for c in range(NCH):
            off = base + c * CH
            pltpu.sync_copy(rel_hbm.at[pl.ds(off, CH)], idx_v)
            gather = pltpu.async_copy(table_hbm.at[idx_v], r_v, sem)
            pltpu.sync_copy(s_hbm.at[pl.ds(off * D, CH * D)], s_v)
            pltpu.sync_copy(o_hbm.at[pl.ds(off * D, CH * D)], o_v)
            gather.wait()

            def group_body(g, _, c=c):
                acc = jnp.zeros((L,), jnp.float32)
                for i in range(L):
                    row = g * L + i
                    rowvec = jnp.zeros((L,), jnp.float32)
                    for k in range(KD):
                        sv = s_v[pl.ds(row * D + k * L, L)]
                        ov = o_v[pl.ds(row * D + k * L, L)]
                        rv = r_v[row, pl.ds(k * L, L)]
                        rowvec = rowvec + sv * rv * ov
                    acc = jnp.where(lanes == i, jnp.sum(rowvec), acc)
                out_v[pl.ds(c * CH + g * L, L)] = acc
                return 0

            lax.fori_loop(0, CH // L, group_body, 0)

        pltpu.sync_copy(out_v, out_hbm.at[pl.ds(base, BPW)])

    return sc_kernel


_sc_kernel = _make_sc_kernel()


def kernel(subject_embeddings, object_embeddings, relations, relation_weight):
    scores = _sc_kernel(subject_embeddings.reshape(B * D),
                        object_embeddings.reshape(B * D),
                        relations.astype(jnp.int32),
                        relation_weight)
    return scores.reshape(B, 1)


# speedup vs baseline: 1.6996x; 1.1579x over previous
import functools

import jax
import jax.numpy as jnp
from jax import lax
from jax.experimental import pallas as pl
from jax.experimental.pallas import tpu as pltpu
from jax.experimental.pallas import tpu_sc as plsc

B = 16384
D = 64
DP = 128
NUM_REL = 1000

_info = plsc.get_sparse_core_info()
NC = _info.num_cores
NS = _info.num_subcores
L = _info.num_lanes
NW = NC * NS
BPW = B // NW
CH = 128
NCH = BPW // CH
KD = D // L


def _make_sc_kernel():
    mesh = plsc.VectorSubcoreMesh(core_axis_name="c", subcore_axis_name="s")

    @functools.partial(
        pl.kernel,
        mesh=mesh,
        compiler_params=pltpu.CompilerParams(needs_layout_passes=False,
                                             use_tc_tiling_on_sc=True),
        out_type=jax.ShapeDtypeStruct((B,), jnp.float32),
        scratch_types=[
            pltpu.VMEM((CH,), jnp.int32),
            pltpu.VMEM((CH, D), jnp.float32),
            pltpu.VMEM((CH, D), jnp.float32),
            pltpu.VMEM((CH, DP), jnp.float32),
            pltpu.VMEM((BPW,), jnp.float32),
            pltpu.SemaphoreType.DMA,
        ],
    )
    def sc_kernel(s_hbm, o_hbm, rel_hbm, table_hbm, out_hbm,
                  idx_v, s_v, o_v, r_v, out_v, sem):
        wid = lax.axis_index("s") * NC + lax.axis_index("c")
        base = wid * BPW
        lanes = lax.iota(jnp.int32, L)

        for c in range(NCH):
            off = base + c * CH
            pltpu.sync_copy(rel_hbm.at[pl.ds(off, CH)], idx_v)
            gather = pltpu.async_copy(table_hbm.at[idx_v], r_v, sem)
            pltpu.sync_copy(s_hbm.at[pl.ds(off, CH), :], s_v)
            pltpu.sync_copy(o_hbm.at[pl.ds(off, CH), :], o_v)
            gather.wait()

            def group_body(g, _, c=c):
                acc = jnp.zeros((L,), jnp.float32)
                for i in range(L):
                    row = g * L + i
                    rowvec = jnp.zeros((L,), jnp.float32)
                    for k in range(KD):
                        sv = s_v[row, pl.ds(k * L, L)]
                        ov = o_v[row, pl.ds(k * L, L)]
                        rv = r_v[row, pl.ds(k * L, L)]
                        rowvec = rowvec + sv * rv * ov
                    acc = jnp.where(lanes == i, jnp.sum(rowvec), acc)
                out_v[pl.ds(c * CH + g * L, L)] = acc
                return 0

            lax.fori_loop(0, CH // L, group_body, 0)

        pltpu.sync_copy(out_v, out_hbm.at[pl.ds(base, BPW)])

    return sc_kernel


_sc_kernel = _make_sc_kernel()


def kernel(subject_embeddings, object_embeddings, relations, relation_weight):
    table_padded = jnp.pad(relation_weight, ((0, 0), (0, DP - D)))
    scores = _sc_kernel(subject_embeddings, object_embeddings,
                        relations.astype(jnp.int32), table_padded)
    return scores.reshape(B, 1)


# speedup vs baseline: 2.3381x; 1.3757x over previous
import functools

import jax
import jax.numpy as jnp
from jax import lax
from jax.experimental import pallas as pl
from jax.experimental.pallas import tpu as pltpu
from jax.experimental.pallas import tpu_sc as plsc

B = 16384
D = 64
NUM_REL = 1000
TP = 1024

_info = plsc.get_sparse_core_info()
NC = _info.num_cores
NS = _info.num_subcores
L = _info.num_lanes
NW = NC * NS
BPW = B // NW
CH = 128
NCH = BPW // CH


def _make_sc_kernel():
    mesh = plsc.VectorSubcoreMesh(core_axis_name="c", subcore_axis_name="s")

    @functools.partial(
        pl.kernel,
        mesh=mesh,
        compiler_params=pltpu.CompilerParams(needs_layout_passes=False,
                                             use_tc_tiling_on_sc=True),
        out_type=jax.ShapeDtypeStruct((B,), jnp.float32),
        scratch_types=[
            pltpu.VMEM((D * TP,), jnp.float32),
            pltpu.VMEM((BPW,), jnp.int32),
            pltpu.VMEM((D, CH), jnp.float32),
            pltpu.VMEM((D, CH), jnp.float32),
            pltpu.VMEM((BPW,), jnp.float32),
        ],
    )
    def sc_kernel(sT_hbm, oT_hbm, rel_hbm, tflat_hbm, out_hbm,
                  tflat_v, idx_v, sT_v, oT_v, out_v):
        wid = lax.axis_index("s") * NC + lax.axis_index("c")
        base = wid * BPW

        pltpu.sync_copy(tflat_hbm, tflat_v)
        pltpu.sync_copy(rel_hbm.at[pl.ds(base, BPW)], idx_v)

        for c in range(NCH):
            off = base + c * CH
            pltpu.sync_copy(sT_hbm.at[:, pl.ds(off, CH)], sT_v)
            pltpu.sync_copy(oT_hbm.at[:, pl.ds(off, CH)], oT_v)

            def group_body(g, _, c=c):
                idx16 = idx_v[pl.ds(c * CH + g * L, L)]
                acc = jnp.zeros((L,), jnp.float32)
                for d in range(D):
                    sv = sT_v[d, pl.ds(g * L, L)]
                    ov = oT_v[d, pl.ds(g * L, L)]
                    rv = plsc.load_gather(tflat_v, [idx16 + d * TP])
                    acc = acc + sv * rv * ov
                out_v[pl.ds(c * CH + g * L, L)] = acc
                return 0

            lax.fori_loop(0, CH // L, group_body, 0)

        pltpu.sync_copy(out_v, out_hbm.at[pl.ds(base, BPW)])

    return sc_kernel


_sc_kernel = _make_sc_kernel()


def kernel(subject_embeddings, object_embeddings, relations, relation_weight):
    tflat = jnp.pad(relation_weight.T, ((0, 0), (0, TP - NUM_REL))).reshape(D * TP)
    scores = _sc_kernel(subject_embeddings.T, object_embeddings.T,
                        relations.astype(jnp.int32), tflat)
    return scores.reshape(B, 1)


# speedup vs baseline: 2.7676x; 1.1837x over previous
import functools

import jax
import jax.numpy as jnp
from jax import lax
from jax.experimental import pallas as pl
from jax.experimental.pallas import tpu as pltpu
from jax.experimental.pallas import tpu_sc as plsc

B = 16384
D = 64
NUM_REL = 1000
TP = 1024

_info = plsc.get_sparse_core_info()
NC = _info.num_cores
NS = _info.num_subcores
L = _info.num_lanes
NW = NC * NS
BPW = B // NW
CH = 128
NCH = BPW // CH


def _make_sc_kernel():
    mesh = plsc.VectorSubcoreMesh(core_axis_name="c", subcore_axis_name="s")

    @functools.partial(
        pl.kernel,
        mesh=mesh,
        compiler_params=pltpu.CompilerParams(needs_layout_passes=False,
                                             use_tc_tiling_on_sc=True),
        out_type=jax.ShapeDtypeStruct((B,), jnp.float32),
        scratch_types=[
            pltpu.VMEM((D * TP,), jnp.float32),
            pltpu.VMEM((BPW,), jnp.int32),
            pltpu.VMEM((D, CH), jnp.float32),
            pltpu.VMEM((D, CH), jnp.float32),
            pltpu.VMEM((D, CH), jnp.float32),
            pltpu.VMEM((D, CH), jnp.float32),
            pltpu.VMEM((BPW,), jnp.float32),
            pltpu.SemaphoreType.DMA,
            pltpu.SemaphoreType.DMA,
            pltpu.SemaphoreType.DMA,
        ],
    )
    def sc_kernel(sT_hbm, oT_hbm, rel_hbm, tflat_hbm, out_hbm,
                  tflat_v, idx_v, s0_v, o0_v, s1_v, o1_v, out_v,
                  sem_t, sem_b0, sem_b1):
        wid = lax.axis_index("s") * NC + lax.axis_index("c")
        base = wid * BPW
        bufs = ((s0_v, o0_v, sem_b0), (s1_v, o1_v, sem_b1))

        tcopy = pltpu.async_copy(tflat_hbm, tflat_v, sem_t)
        pltpu.sync_copy(rel_hbm.at[pl.ds(base, BPW)], idx_v)

        def prefetch(c):
            s_v, o_v, sem = bufs[c % 2]
            off = base + c * CH
            return (pltpu.async_copy(sT_hbm.at[:, pl.ds(off, CH)], s_v, sem),
                    pltpu.async_copy(oT_hbm.at[:, pl.ds(off, CH)], o_v, sem))

        pending = prefetch(0)
        for c in range(NCH):
            s_v, o_v, _ = bufs[c % 2]
            nxt = prefetch(c + 1) if c + 1 < NCH else ()
            for h in pending:
                h.wait()
            pending = nxt
            if c == 0:
                tcopy.wait()

            def group_body(g, _, c=c, s_v=s_v, o_v=o_v):
                idx16 = idx_v[pl.ds(c * CH + g * L, L)]
                acc = jnp.zeros((L,), jnp.float32)
                for d in range(D):
                    sv = s_v[d, pl.ds(g * L, L)]
                    ov = o_v[d, pl.ds(g * L, L)]
                    rv = plsc.load_gather(tflat_v, [idx16 + d * TP])
                    acc = acc + sv * rv * ov
                out_v[pl.ds(c * CH + g * L, L)] = acc
                return 0

            lax.fori_loop(0, CH // L, group_body, 0)

        pltpu.sync_copy(out_v, out_hbm.at[pl.ds(base, BPW)])

    return sc_kernel


_sc_kernel = _make_sc_kernel()


def kernel(subject_embeddings, object_embeddings, relations, relation_weight):
    tflat = jnp.pad(relation_weight.T, ((0, 0), (0, TP - NUM_REL))).reshape(D * TP)
    scores = _sc_kernel(subject_embeddings.T, object_embeddings.T,
                        relations.astype(jnp.int32), tflat)
    return scores.reshape(B, 1)


# speedup vs baseline: 2.9506x; 1.0661x over previous
import functools

import jax
import jax.numpy as jnp
from jax import lax
from jax.experimental import pallas as pl
from jax.experimental.pallas import tpu as pltpu
from jax.experimental.pallas import tpu_sc as plsc

B = 16384
D = 64
NUM_REL = 1000
TP = 1024

_info = plsc.get_sparse_core_info()
NC = _info.num_cores
NS = _info.num_subcores
L = _info.num_lanes
NW = NC * NS
BPW = B // NW
CH = 128
NCH = BPW // CH
DU = 16


def _make_sc_kernel():
    mesh = plsc.VectorSubcoreMesh(core_axis_name="c", subcore_axis_name="s")

    @functools.partial(
        pl.kernel,
        mesh=mesh,
        compiler_params=pltpu.CompilerParams(needs_layout_passes=False,
                                             use_tc_tiling_on_sc=True),
        out_type=jax.ShapeDtypeStruct((B,), jnp.float32),
        scratch_types=[
            pltpu.VMEM((D * TP,), jnp.float32),
            pltpu.VMEM((BPW,), jnp.int32),
            pltpu.VMEM((D, CH), jnp.float32),
            pltpu.VMEM((D, CH), jnp.float32),
            pltpu.VMEM((D, CH), jnp.float32),
            pltpu.VMEM((D, CH), jnp.float32),
            pltpu.VMEM((BPW,), jnp.float32),
            pltpu.SemaphoreType.DMA,
            pltpu.SemaphoreType.DMA,
            pltpu.SemaphoreType.DMA,
        ],
    )
    def sc_kernel(sT_hbm, oT_hbm, rel_hbm, tflat_hbm, out_hbm,
                  tflat_v, idx_v, s0_v, o0_v, s1_v, o1_v, out_v,
                  sem_t, sem_b0, sem_b1):
        wid = lax.axis_index("s") * NC + lax.axis_index("c")
        base = wid * BPW
        bufs = ((s0_v, o0_v, sem_b0), (s1_v, o1_v, sem_b1))

        tcopy = pltpu.async_copy(tflat_hbm, tflat_v, sem_t)
        pltpu.sync_copy(rel_hbm.at[pl.ds(base, BPW)], idx_v)

        def prefetch(c):
            s_v, o_v, sem = bufs[c % 2]
            off = base + c * CH
            return (pltpu.async_copy(sT_hbm.at[:, pl.ds(off, CH)], s_v, sem),
                    pltpu.async_copy(oT_hbm.at[:, pl.ds(off, CH)], o_v, sem))

        pending = prefetch(0)
        for c in range(NCH):
            s_v, o_v, _ = bufs[c % 2]
            nxt = prefetch(c + 1) if c + 1 < NCH else ()
            for h in pending:
                h.wait()
            pending = nxt
            if c == 0:
                tcopy.wait()

            def group_body(g, _, c=c, s_v=s_v, o_v=o_v):
                idx16 = idx_v[pl.ds(c * CH + g * L, L)]

                def dq_body(dq, acc, g=g, idx16=idx16, s_v=s_v, o_v=o_v):
                    d0 = dq * DU
                    for j in range(DU):
                        sv = s_v[d0 + j, pl.ds(g * L, L)]
                        ov = o_v[d0 + j, pl.ds(g * L, L)]
                        rv = plsc.load_gather(tflat_v, [idx16 + (d0 + j) * TP])
                        acc = acc + sv * rv * ov
                    return acc

                acc = lax.fori_loop(0, D // DU, dq_body,
                                    jnp.zeros((L,), jnp.float32))
                out_v[pl.ds(c * CH + g * L, L)] = acc
                return 0

            lax.fori_loop(0, CH // L, group_body, 0)

        pltpu.sync_copy(out_v, out_hbm.at[pl.ds(base, BPW)])

    return sc_kernel


_sc_kernel = _make_sc_kernel()


def kernel(subject_embeddings, object_embeddings, relations, relation_weight):
    tflat = jnp.pad(relation_weight.T, ((0, 0), (0, TP - NUM_REL))).reshape(D * TP)
    scores = _sc_kernel(subject_embeddings.T, object_embeddings.T,
                        relations.astype(jnp.int32), tflat)
    return scores.reshape(B, 1)


# speedup vs baseline: 3.3804x; 1.1457x over previous
import functools

import jax
import jax.numpy as jnp
from jax import lax
from jax.experimental import pallas as pl
from jax.experimental.pallas import tpu as pltpu
from jax.experimental.pallas import tpu_sc as plsc

B = 16384
D = 64
NUM_REL = 1000
TP = 1024

_info = plsc.get_sparse_core_info()
NC = _info.num_cores
NS = _info.num_subcores
L = _info.num_lanes
NW = NC * NS
BPW = B // NW
CH = 128
NCH = BPW // CH
DU = 16


def _make_sc_kernel():
    mesh = plsc.VectorSubcoreMesh(core_axis_name="c", subcore_axis_name="s")

    @functools.partial(
        pl.kernel,
        mesh=mesh,
        compiler_params=pltpu.CompilerParams(needs_layout_passes=False,
                                             use_tc_tiling_on_sc=True),
        out_type=jax.ShapeDtypeStruct((B,), jnp.float32),
        scratch_types=[
            pltpu.VMEM((D * TP,), jnp.float32),
            pltpu.VMEM((BPW,), jnp.int32),
            pltpu.VMEM((D, CH), jnp.float32),
            pltpu.VMEM((D, CH), jnp.float32),
            pltpu.VMEM((D, CH), jnp.float32),
            pltpu.VMEM((D, CH), jnp.float32),
            pltpu.VMEM((BPW,), jnp.float32),
            pltpu.VMEM_SHARED((D * TP,), jnp.float32),
            pltpu.SemaphoreType.DMA,
            pltpu.SemaphoreType.DMA,
        ],
    )
    def sc_kernel(sT_hbm, oT_hbm, rel_hbm, tflat_hbm, out_hbm,
                  tflat_v, idx_v, s0_v, o0_v, s1_v, o1_v, out_v,
                  shared_v, sem_b0, sem_b1):
        sid = lax.axis_index("s")
        wid = sid * NC + lax.axis_index("c")
        base = wid * BPW
        bufs = ((s0_v, o0_v, sem_b0), (s1_v, o1_v, sem_b1))

        def prefetch(c):
            s_v, o_v, sem = bufs[c % 2]
            off = base + c * CH
            return (pltpu.async_copy(sT_hbm.at[:, pl.ds(off, CH)], s_v, sem),
                    pltpu.async_copy(oT_hbm.at[:, pl.ds(off, CH)], o_v, sem))

        pending = prefetch(0)
        pltpu.sync_copy(rel_hbm.at[pl.ds(base, BPW)], idx_v)

        @pl.when(sid == 0)
        def _():
            pltpu.sync_copy(tflat_hbm, shared_v)
        plsc.subcore_barrier()
        pltpu.sync_copy(shared_v, tflat_v)

        for c in range(NCH):
            s_v, o_v, _ = bufs[c % 2]
            nxt = prefetch(c + 1) if c + 1 < NCH else ()
            for h in pending:
                h.wait()
            pending = nxt

            def group_body(g, _, c=c, s_v=s_v, o_v=o_v):
                idx16 = idx_v[pl.ds(c * CH + g * L, L)]

                def dq_body(dq, accs, g=g, idx16=idx16, s_v=s_v, o_v=o_v):
                    d0 = dq * DU
                    accs = list(accs)
                    for j in range(DU):
                        sv = s_v[d0 + j, pl.ds(g * L, L)]
                        ov = o_v[d0 + j, pl.ds(g * L, L)]
                        rv = plsc.load_gather(tflat_v, [idx16 + (d0 + j) * TP])
                        accs[j % 4] = accs[j % 4] + sv * rv * ov
                    return tuple(accs)

                z = jnp.zeros((L,), jnp.float32)
                a0, a1, a2, a3 = lax.fori_loop(0, D // DU, dq_body,
                                               (z, z, z, z))
                out_v[pl.ds(c * CH + g * L, L)] = (a0 + a1) + (a2 + a3)
                return 0

            lax.fori_loop(0, CH // L, group_body, 0)

        pltpu.sync_copy(out_v, out_hbm.at[pl.ds(base, BPW)])

    return sc_kernel


_sc_kernel = _make_sc_kernel()


def kernel(subject_embeddings, object_embeddings, relations, relation_weight):
    tflat = jnp.pad(relation_weight.T, ((0, 0), (0, TP - NUM_REL))).reshape(D * TP)
    scores = _sc_kernel(subject_embeddings.T, object_embeddings.T,
                        relations.astype(jnp.int32), tflat)
    return scores.reshape(B, 1)


# speedup vs baseline: 3.4305x; 1.0148x over previous
import functools

import jax
import jax.numpy as jnp
from jax import lax
from jax.experimental import pallas as pl
from jax.experimental.pallas import tpu as pltpu
from jax.experimental.pallas import tpu_sc as plsc

B = 16384
D = 64
NUM_REL = 1000
TP = 1024

_info = plsc.get_sparse_core_info()
NC = _info.num_cores
NS = _info.num_subcores
L = _info.num_lanes
NW = NC * NS
BPW = B // NW
CH = 128
NCH = BPW // CH
DU = 8


def _make_sc_kernel():
    mesh = plsc.VectorSubcoreMesh(core_axis_name="c", subcore_axis_name="s")

    @functools.partial(
        pl.kernel,
        mesh=mesh,
        compiler_params=pltpu.CompilerParams(needs_layout_passes=False,
                                             use_tc_tiling_on_sc=True),
        out_type=jax.ShapeDtypeStruct((B,), jnp.float32),
        scratch_types=[
            pltpu.VMEM((D * TP,), jnp.float32),
            pltpu.VMEM((BPW,), jnp.int32),
            pltpu.VMEM((D, CH), jnp.float32),
            pltpu.VMEM((D, CH), jnp.float32),
            pltpu.VMEM((D, CH), jnp.float32),
            pltpu.VMEM((D, CH), jnp.float32),
            pltpu.VMEM((BPW,), jnp.float32),
            pltpu.VMEM_SHARED((D * TP,), jnp.float32),
            pltpu.SemaphoreType.DMA,
            pltpu.SemaphoreType.DMA,
        ],
    )
    def sc_kernel(sT_hbm, oT_hbm, rel_hbm, tflat_hbm, out_hbm,
                  tflat_v, idx_v, s0_v, o0_v, s1_v, o1_v, out_v,
                  shared_v, sem_b0, sem_b1):
        sid = lax.axis_index("s")
        wid = sid * NC + lax.axis_index("c")
        base = wid * BPW
        bufs = ((s0_v, o0_v, sem_b0), (s1_v, o1_v, sem_b1))

        def prefetch(c):
            s_v, o_v, sem = bufs[c % 2]
            off = base + c * CH
            return (pltpu.async_copy(sT_hbm.at[:, pl.ds(off, CH)], s_v, sem),
                    pltpu.async_copy(oT_hbm.at[:, pl.ds(off, CH)], o_v, sem))

        pending = prefetch(0)
        pltpu.sync_copy(rel_hbm.at[pl.ds(base, BPW)], idx_v)

        @pl.when(sid == 0)
        def _():
            pltpu.sync_copy(tflat_hbm, shared_v)
        plsc.subcore_barrier()
        pltpu.sync_copy(shared_v, tflat_v)

        for c in range(NCH):
            s_v, o_v, _ = bufs[c % 2]
            nxt = prefetch(c + 1) if c + 1 < NCH else ()
            for h in pending:
                h.wait()
            pending = nxt

            def group_body(g, _, c=c, s_v=s_v, o_v=o_v):
                idx16 = idx_v[pl.ds(c * CH + g * L, L)]

                def dq_body(dq, accs, g=g, idx16=idx16, s_v=s_v, o_v=o_v):
                    d0 = dq * DU
                    accs = list(accs)
                    for j in range(DU):
                        sv = s_v[d0 + j, pl.ds(g * L, L)]
                        ov = o_v[d0 + j, pl.ds(g * L, L)]
                        rv = plsc.load_gather(tflat_v, [idx16 + (d0 + j) * TP])
                        accs[j % 4] = accs[j % 4] + sv * rv * ov
                    return tuple(accs)

                z = jnp.zeros((L,), jnp.float32)
                a0, a1, a2, a3 = lax.fori_loop(0, D // DU, dq_body,
                                               (z, z, z, z))
                out_v[pl.ds(c * CH + g * L, L)] = (a0 + a1) + (a2 + a3)
                return 0

            lax.fori_loop(0, CH // L, group_body, 0)

        pltpu.sync_copy(out_v, out_hbm.at[pl.ds(base, BPW)])

    return sc_kernel


_sc_kernel = _make_sc_kernel()


def kernel(subject_embeddings, object_embeddings, relations, relation_weight):
    tflat = jnp.pad(relation_weight.T, ((0, 0), (0, TP - NUM_REL))).reshape(D * TP)
    scores = _sc_kernel(subject_embeddings.T, object_embeddings.T,
                        relations.astype(jnp.int32), tflat)
    return scores.reshape(B, 1)
